# baseline (speedup 1.0000x reference)
import functools
import jax
import jax.numpy as jnp
from jax import lax
from jax.experimental import pallas as pl
from jax.experimental.pallas import tpu as pltpu
from jax.experimental.pallas import tpu_sc as plsc

NUM_ROWS = 128
ROW_LEN = 100000
LANES = 16
NUM_CORES = 2
NUM_SUBCORES = 16
NUM_WORKERS = NUM_CORES * NUM_SUBCORES

GROUP_ROWS = 8
NUM_GROUPS = NUM_ROWS // GROUP_ROWS
HALF = 49920
SC_LEN = 2 * HALF
TAIL = ROW_LEN - SC_LEN

CHUNK_COLS = 3328
CHUNKS = HALF // CHUNK_COLS
STEPS = CHUNK_COLS // LANES


def _merge(a, b):
    mv_a, mi_a = a
    mv_b, mi_b = b
    better = (mv_b > mv_a) | ((mv_b == mv_a) & (mi_b < mi_a))
    return (jnp.where(better, mv_b, mv_a), jnp.where(better, mi_b, mi_a))


def _sc_body(logits_hbm, val_hbm, idx_hbm, buf0, buf1, vres_ref, ires_ref,
             sem0, sem1):
    c = lax.axis_index("c")
    s = lax.axis_index("s")
    wid = c * NUM_SUBCORES + s
    group = wid // 2
    half = wid % 2
    row0 = group * GROUP_ROWS
    col0 = half * HALF

    bufs = (buf0, buf1)
    sems = (sem0, sem1)

    def start(g):
        off = col0 + g * CHUNK_COLS
        return pltpu.async_copy(
            logits_hbm.at[pl.ds(row0, GROUP_ROWS), pl.ds(off, CHUNK_COLS)],
            bufs[g % 2],
            sems[g % 2],
        )

    lane_iota = lax.iota(jnp.int32, LANES)
    neg_inf = jnp.full((LANES,), -jnp.inf, jnp.float32)
    zeros_i = jnp.zeros((LANES,), jnp.int32)

    pending = start(0)
    accs = tuple((neg_inf, zeros_i) for _ in range(GROUP_ROWS))

    for g in range(CHUNKS):
        cbuf = bufs[g % 2]
        nxt = start(g + 1) if g + 1 < CHUNKS else None
        pending.wait()
        pending = nxt

        chunk_base = g * STEPS

        @plsc.parallel_loop(0, STEPS, carry=accs, unroll=2)
        def accs(j, carry):
            jv = jnp.full((LANES,), chunk_base + j, jnp.int32)
            new = []
            for r in range(GROUP_ROWS):
                mv, mi = carry[r]
                v = cbuf[r, pl.ds(j * LANES, LANES)]
                gt = v > mv
                mv = jnp.maximum(mv, v)
                mi = jnp.where(gt, jv, mi)
                new.append((mv, mi))
            return tuple(new)

    vres_vec = jnp.zeros((LANES,), jnp.float32)
    ires_vec = jnp.zeros((LANES,), jnp.int32)
    for r in range(GROUP_ROWS):
        m, mi = accs[r]
        full_idx = mi * LANES + lane_iota
        bv = m[0]
        bi = full_idx[0]
        for l in range(1, LANES):
            v = m[l]
            fi = full_idx[l]
            better = (v > bv) | ((v == bv) & (fi < bi))
            bv = jnp.where(better, v, bv)
            bi = jnp.where(better, fi, bi)
        sel = lane_iota == r
        vres_vec = jnp.where(sel, jnp.full((LANES,), bv, jnp.float32), vres_vec)
        ires_vec = jnp.where(
            sel, jnp.full((LANES,), bi + col0, jnp.int32), ires_vec
        )

    vres_ref[...] = vres_vec
    ires_ref[...] = ires_vec
    pltpu.sync_copy(vres_ref, val_hbm.at[wid])
    pltpu.sync_copy(ires_ref, idx_hbm.at[wid])


def _tc_body(tail_ref, av_ref, ai_ref, bv_ref, bi_ref, out_ref):
    t = tail_ref[...]
    col = lax.broadcasted_iota(jnp.int32, (NUM_ROWS, TAIL), 1)
    tmax = jnp.max(t, axis=1, keepdims=True)
    cand = jnp.where(t == tmax, col, TAIL)
    targ = jnp.min(cand, axis=1, keepdims=True) + SC_LEN

    av, ai = av_ref[...], ai_ref[...]
    bv, bi = bv_ref[...], bi_ref[...]
    take_b = bv > av
    mv = jnp.where(take_b, bv, av)
    mi = jnp.where(take_b, bi, ai)
    out_ref[...] = jnp.where(tmax > mv, targ, mi)


@jax.jit
def _argmax_impl(logits):
    mesh = plsc.VectorSubcoreMesh(core_axis_name="c", subcore_axis_name="s")
    sc = pl.kernel(
        _sc_body,
        out_type=(
            jax.ShapeDtypeStruct((NUM_WORKERS, LANES), jnp.float32),
            jax.ShapeDtypeStruct((NUM_WORKERS, LANES), jnp.int32),
        ),
        mesh=mesh,
        scratch_types=[
            pltpu.VMEM((GROUP_ROWS, CHUNK_COLS), jnp.float32),
            pltpu.VMEM((GROUP_ROWS, CHUNK_COLS), jnp.float32),
            pltpu.VMEM((LANES,), jnp.float32),
            pltpu.VMEM((LANES,), jnp.int32),
            pltpu.SemaphoreType.DMA,
            pltpu.SemaphoreType.DMA,
        ],
    )
    vals, idxs = sc(logits)
    v = vals[:, :GROUP_ROWS].reshape(NUM_GROUPS, 2, GROUP_ROWS)
    i = idxs[:, :GROUP_ROWS].reshape(NUM_GROUPS, 2, GROUP_ROWS)
    av = v[:, 0, :].reshape(NUM_ROWS, 1)
    bv = v[:, 1, :].reshape(NUM_ROWS, 1)
    ai = i[:, 0, :].reshape(NUM_ROWS, 1)
    bi = i[:, 1, :].reshape(NUM_ROWS, 1)

    tail = lax.slice(logits, (0, SC_LEN), (NUM_ROWS, ROW_LEN))
    out = pl.pallas_call(
        _tc_body,
        out_shape=jax.ShapeDtypeStruct((NUM_ROWS, 1), jnp.int32),
    )(tail, av, ai, bv, bi)
    return out.reshape(NUM_ROWS)


def kernel(logits, temperatures):
    return _argmax_impl(logits)


# speedup vs baseline: 1.3418x; 1.3418x over previous
import jax
import jax.numpy as jnp
from jax import lax
from jax.experimental import pallas as pl
from jax.experimental.pallas import tpu as pltpu

NUM_ROWS = 128
ROW_LEN = 100000
BLOCK_COLS = 8192
NUM_BLOCKS = (ROW_LEN + BLOCK_COLS - 1) // BLOCK_COLS


def _body(x_ref, out_ref, vmax_ref, vidx_ref):
    j = pl.program_id(0)

    @pl.when(j == 0)
    def _():
        vmax_ref[...] = jnp.full((NUM_ROWS, 1), -jnp.inf, jnp.float32)
        vidx_ref[...] = jnp.zeros((NUM_ROWS, 1), jnp.int32)

    t = x_ref[...]
    col = lax.broadcasted_iota(jnp.int32, (NUM_ROWS, BLOCK_COLS), 1)
    colg = col + j * BLOCK_COLS
    t = jnp.where(colg < ROW_LEN, t, -jnp.inf)

    bmax = jnp.max(t, axis=1, keepdims=True)
    cand = jnp.where(t == bmax, colg, ROW_LEN)
    barg = jnp.min(cand, axis=1, keepdims=True)

    better = bmax > vmax_ref[...]
    vmax_ref[...] = jnp.where(better, bmax, vmax_ref[...])
    vidx_ref[...] = jnp.where(better, barg, vidx_ref[...])

    @pl.when(j == NUM_BLOCKS - 1)
    def _():
        out_ref[...] = vidx_ref[...]


@jax.jit
def _argmax_impl(logits):
    out = pl.pallas_call(
        _body,
        grid=(NUM_BLOCKS,),
        in_specs=[
            pl.BlockSpec((NUM_ROWS, BLOCK_COLS), lambda j: (0, j)),
        ],
        out_specs=pl.BlockSpec((NUM_ROWS, 1), lambda j: (0, 0)),
        out_shape=jax.ShapeDtypeStruct((NUM_ROWS, 1), jnp.int32),
        scratch_shapes=[
            pltpu.VMEM((NUM_ROWS, 1), jnp.float32),
            pltpu.VMEM((NUM_ROWS, 1), jnp.int32),
        ],
    )(logits)
    return out.reshape(NUM_ROWS)


def kernel(logits, temperatures):
    return _argmax_impl(logits)
